# baseline (speedup 1.0000x reference)
import functools
import math

import numpy as np

import jax
import jax.numpy as jnp
from jax import lax
from jax.experimental import pallas as pl
from jax.experimental.pallas import tpu as pltpu
from jax.experimental.pallas import tpu_sc as plsc

_NC = 2
_NS = 16
_NW = _NC * _NS


def _sc_gather(table, seqs):
    b, s = seqs.shape
    n = b * s
    _, d = table.shape
    b_per_w = n // _NW
    chunk = 32
    n_chunks = b_per_w // chunk
    pairs = n_chunks // 2
    w_per_row = s // b_per_w
    mesh = plsc.VectorSubcoreMesh(core_axis_name="c", subcore_axis_name="s")

    @functools.partial(
        pl.kernel,
        mesh=mesh,
        out_type=jax.ShapeDtypeStruct((n, d), jnp.float32),
        scratch_types=[
            pltpu.VMEM((b_per_w,), jnp.int32),
            pltpu.VMEM((chunk, d), jnp.float32),
            pltpu.VMEM((chunk, d), jnp.float32),
            pltpu.SemaphoreType.DMA,
            pltpu.SemaphoreType.DMA,
            pltpu.SemaphoreType.DMA,
            pltpu.SemaphoreType.DMA,
        ],
    )
    def gather_kernel(table_hbm, idx_hbm, out_hbm, idx_v, bufa, bufb,
                      gsa, gsb, wsa, wsb):
        wid = lax.axis_index("s") * _NC + lax.axis_index("c")
        base = wid * b_per_w

        def gather_c(c, buf, sem):
            return pltpu.make_async_copy(
                table_hbm.at[idx_v.at[pl.ds(c * chunk, chunk)]], buf, sem)

        def write_c(c, buf, sem):
            return pltpu.make_async_copy(
                buf, out_hbm.at[pl.ds(base + c * chunk, chunk)], sem)

        pltpu.sync_copy(
            idx_hbm.at[wid // w_per_row,
                       pl.ds((wid % w_per_row) * b_per_w, b_per_w)], idx_v)
        gather_c(0, bufa, gsa).start()

        @pl.loop(0, pairs)
        def _(p):
            a = 2 * p
            bb = a + 1
            gather_c(a, bufa, gsa).wait()

            @pl.when(p > 0)
            def _():
                write_c(bb - 2, bufb, wsb).wait()

            gather_c(bb, bufb, gsb).start()
            write_c(a, bufa, wsa).start()
            gather_c(bb, bufb, gsb).wait()
            write_c(bb, bufb, wsb).start()

            @pl.when(p < pairs - 1)
            def _():
                write_c(a, bufa, wsa).wait()
                gather_c(a + 2, bufa, gsa).start()

        write_c(n_chunks - 2, bufa, wsa).wait()
        write_c(n_chunks - 1, bufb, wsb).wait()

    return gather_kernel(table, seqs)


@functools.lru_cache(maxsize=None)
def _pos_encoding(seq_len, dim):
    pos = np.arange(seq_len, dtype=np.float64)[:, None]
    i = np.arange(dim // 2, dtype=np.float64)[None, :]
    angle = (pos / np.power(10000.0, 2.0 * i / dim)).astype(np.float32)
    pe = np.stack([np.sin(angle), np.cos(angle)], axis=-1).reshape(
        seq_len, dim).astype(np.float32)
    return jnp.asarray(pe)


def _tc_scale_pe_ln_chunk(g3, pe, ln_weight, ln_bias, full_b, batch0, prev):
    bc, s, d = g3.shape
    sb = 512
    scale = math.sqrt(float(d))

    def ln_kernel(x_ref, pe_ref, w_ref, b_ref, *rest):
        o_ref = rest[-1]
        x = x_ref[...] * scale + pe_ref[...][None, :, :]
        m = jnp.mean(x, axis=2, keepdims=True)
        xc = x - m
        var = jnp.mean(xc * xc, axis=2, keepdims=True)
        o_ref[...] = (xc * lax.rsqrt(var + 1e-5)) * w_ref[...] + b_ref[...]

    in_specs = [
        pl.BlockSpec((bc, sb, d), lambda j: (0, j, 0)),
        pl.BlockSpec((sb, d), lambda j: (j, 0)),
        pl.BlockSpec((1, 1, d), lambda j: (0, 0, 0)),
        pl.BlockSpec((1, 1, d), lambda j: (0, 0, 0)),
    ]
    args = [g3, pe, ln_weight.reshape(1, 1, d), ln_bias.reshape(1, 1, d)]
    io_alias = {}
    if prev is not None:
        in_specs.append(pl.BlockSpec(memory_space=pltpu.MemorySpace.HBM))
        args.append(prev)
        io_alias = {4: 0}
    cb = batch0 // bc
    return pl.pallas_call(
        ln_kernel,
        grid=(s // sb,),
        in_specs=in_specs,
        out_specs=pl.BlockSpec((bc, sb, d), lambda j: (cb, j, 0)),
        out_shape=jax.ShapeDtypeStruct((full_b, s, d), jnp.float32),
        input_output_aliases=io_alias,
    )(*args)


_NSPLIT = 2


def kernel(seqs, padding_mask, embed_table, ln_weight, ln_bias):
    b, s = seqs.shape
    _, d = embed_table.shape
    pe = _pos_encoding(s, d)
    bc = b // _NSPLIT
    out = None
    for c in range(_NSPLIT):
        g = _sc_gather(embed_table, seqs[c * bc:(c + 1) * bc])
        out = _tc_scale_pe_ln_chunk(g.reshape(bc, s, d), pe, ln_weight,
                                    ln_bias, b, c * bc, out)
    return out, padding_mask


# speedup vs baseline: 1.0671x; 1.0671x over previous
import functools
import math

import numpy as np

import jax
import jax.numpy as jnp
from jax import lax
from jax.experimental import pallas as pl
from jax.experimental.pallas import tpu as pltpu
from jax.experimental.pallas import tpu_sc as plsc

_NC = 2
_NS = 16
_NW = _NC * _NS


def _sc_gather(table, seqs):
    b, s = seqs.shape
    n = b * s
    _, d = table.shape
    b_per_w = n // _NW
    chunk = 32
    n_chunks = b_per_w // chunk
    pairs = n_chunks // 2
    w_per_row = s // b_per_w
    mesh = plsc.VectorSubcoreMesh(core_axis_name="c", subcore_axis_name="s")

    @functools.partial(
        pl.kernel,
        mesh=mesh,
        out_type=jax.ShapeDtypeStruct((n, d), jnp.float32),
        scratch_types=[
            pltpu.VMEM((b_per_w,), jnp.int32),
            pltpu.VMEM((chunk, d), jnp.float32),
            pltpu.VMEM((chunk, d), jnp.float32),
            pltpu.SemaphoreType.DMA,
            pltpu.SemaphoreType.DMA,
            pltpu.SemaphoreType.DMA,
            pltpu.SemaphoreType.DMA,
        ],
    )
    def gather_kernel(table_hbm, idx_hbm, out_hbm, idx_v, bufa, bufb,
                      gsa, gsb, wsa, wsb):
        wid = lax.axis_index("s") * _NC + lax.axis_index("c")
        base = wid * b_per_w

        def gather_c(c, buf, sem):
            return pltpu.make_async_copy(
                table_hbm.at[idx_v.at[pl.ds(c * chunk, chunk)]], buf, sem)

        def write_c(c, buf, sem):
            return pltpu.make_async_copy(
                buf, out_hbm.at[pl.ds(base + c * chunk, chunk)], sem)

        pltpu.sync_copy(
            idx_hbm.at[wid // w_per_row,
                       pl.ds((wid % w_per_row) * b_per_w, b_per_w)], idx_v)
        gather_c(0, bufa, gsa).start()

        @pl.loop(0, pairs)
        def _(p):
            a = 2 * p
            bb = a + 1
            gather_c(a, bufa, gsa).wait()

            @pl.when(p > 0)
            def _():
                write_c(bb - 2, bufb, wsb).wait()

            gather_c(bb, bufb, gsb).start()
            write_c(a, bufa, wsa).start()
            gather_c(bb, bufb, gsb).wait()
            write_c(bb, bufb, wsb).start()

            @pl.when(p < pairs - 1)
            def _():
                write_c(a, bufa, wsa).wait()
                gather_c(a + 2, bufa, gsa).start()

        write_c(n_chunks - 2, bufa, wsa).wait()
        write_c(n_chunks - 1, bufb, wsb).wait()

    return gather_kernel(table, seqs)


@functools.lru_cache(maxsize=None)
def _pos_encoding(seq_len, dim):
    pos = np.arange(seq_len, dtype=np.float64)[:, None]
    i = np.arange(dim // 2, dtype=np.float64)[None, :]
    angle = (pos / np.power(10000.0, 2.0 * i / dim)).astype(np.float32)
    return np.stack([np.sin(angle), np.cos(angle)], axis=-1).reshape(
        seq_len, dim).astype(np.float32)


def _tc_scale_pe_ln_chunk(g3, pe_c, ln_weight, ln_bias, full_s, seq0, prev):
    b, sc, d = g3.shape
    sb = 512
    scale = math.sqrt(float(d))

    def ln_kernel(x_ref, pe_ref, w_ref, b_ref, *rest):
        o_ref = rest[-1]
        x = x_ref[...] * scale + pe_ref[...][None, :, :]
        m = jnp.mean(x, axis=2, keepdims=True)
        xc = x - m
        var = jnp.mean(xc * xc, axis=2, keepdims=True)
        o_ref[...] = (xc * lax.rsqrt(var + 1e-5)) * w_ref[...] + b_ref[...]

    in_specs = [
        pl.BlockSpec((b, sb, d), lambda j: (0, j, 0)),
        pl.BlockSpec((sb, d), lambda j: (j, 0)),
        pl.BlockSpec((1, 1, d), lambda j: (0, 0, 0)),
        pl.BlockSpec((1, 1, d), lambda j: (0, 0, 0)),
    ]
    args = [g3, pe_c, ln_weight.reshape(1, 1, d), ln_bias.reshape(1, 1, d)]
    io_alias = {}
    if prev is not None:
        in_specs.append(pl.BlockSpec(memory_space=pltpu.MemorySpace.HBM))
        args.append(prev)
        io_alias = {4: 0}
    j0 = seq0 // sb
    return pl.pallas_call(
        ln_kernel,
        grid=(sc // sb,),
        in_specs=in_specs,
        out_specs=pl.BlockSpec((b, sb, d), lambda j: (0, j0 + j, 0)),
        out_shape=jax.ShapeDtypeStruct((b, full_s, d), jnp.float32),
        input_output_aliases=io_alias,
    )(*args)


_NSPLIT = 4


def kernel(seqs, padding_mask, embed_table, ln_weight, ln_bias):
    b, s = seqs.shape
    _, d = embed_table.shape
    pe = _pos_encoding(s, d)
    sc = s // _NSPLIT
    out = None
    for c in range(_NSPLIT):
        g = _sc_gather(embed_table, seqs[:, c * sc:(c + 1) * sc])
        out = _tc_scale_pe_ln_chunk(g.reshape(b, sc, d),
                                    jnp.asarray(pe[c * sc:(c + 1) * sc]),
                                    ln_weight, ln_bias, s, c * sc, out)
    return out, padding_mask
